# baseline (speedup 1.0000x reference)
import jax
import jax.numpy as jnp
from jax import lax
from jax.experimental import pallas as pl
from jax.experimental.pallas import tpu as pltpu

N = 36864
D = 64
C = 1024
R = 4096
NB = N // R
BETA = 0.25


def _vq_body(z_ref, cb_ref, zq_ref, idx_ref, bc_ref, loss_ref,
             cbn_ref, baug_ref):
    i = pl.program_id(0)

    @pl.when(i == 0)
    def _init():
        cb = cb_ref[...]
        n = jnp.sqrt(jnp.sum(cb * cb, axis=1, keepdims=True))
        cbn_ref[...] = cb / jnp.maximum(n, 1e-12)
        h = cb.astype(jnp.bfloat16).astype(jnp.float32)
        mid = (cb - h).astype(jnp.bfloat16).astype(jnp.float32)
        baug_ref[:, 0:D] = h
        baug_ref[:, D:2 * D] = mid
        iota = lax.broadcasted_iota(jnp.int32, (C, 32), 0)
        baug_ref[:, 2 * D:2 * D + 32] = jnp.right_shift(iota, 8).astype(
            jnp.float32)
        baug_ref[:, 2 * D + 32:3 * D] = jnp.bitwise_and(iota, 255).astype(
            jnp.float32)
        bc_ref[...] = jnp.zeros_like(bc_ref)
        loss_ref[...] = jnp.zeros_like(loss_ref)

    zb = z_ref[...]
    zn = jnp.sqrt(jnp.sum(zb * zb, axis=1, keepdims=True))
    znorm = zb / jnp.maximum(zn, 1e-12)
    d = lax.dot_general(znorm, cbn_ref[...], (((1,), (1,)), ((), ())),
                        precision=lax.Precision.DEFAULT,
                        preferred_element_type=jnp.float32)
    m = jnp.max(d, axis=1, keepdims=True)
    eqf = jnp.where(d == m, 1.0, 0.0).astype(jnp.float32)
    p = lax.dot_general(eqf, baug_ref[...], (((1,), (0,)), ((), ())),
                        precision=lax.Precision.DEFAULT,
                        preferred_element_type=jnp.float32)
    zq = p[:, 0:D] + p[:, D:2 * D]
    idx_ref[...] = (p[:, 2 * D:2 * D + 1] * 256.0
                    + p[:, 2 * D + 32:2 * D + 33]).astype(jnp.int32)
    zq_ref[...] = zb + (zq - zb)
    diff = zq - zb
    loss_ref[...] = loss_ref[...] + jnp.sum(diff * diff).reshape(1, 1)
    bc_ref[...] = bc_ref[...] + jnp.sum(eqf, axis=0, keepdims=True)


def kernel(z, codebook):
    z_flat = z.reshape(N, D)
    zq_flat, idxo, bc, loss_sum = pl.pallas_call(
        _vq_body,
        grid=(NB,),
        in_specs=[
            pl.BlockSpec((R, D), lambda i: (i, 0)),
            pl.BlockSpec((C, D), lambda i: (0, 0)),
        ],
        out_specs=[
            pl.BlockSpec((R, D), lambda i: (i, 0)),
            pl.BlockSpec((R, 1), lambda i: (i, 0)),
            pl.BlockSpec((1, C), lambda i: (0, 0)),
            pl.BlockSpec((1, 1), lambda i: (0, 0)),
        ],
        out_shape=[
            jax.ShapeDtypeStruct((N, D), jnp.float32),
            jax.ShapeDtypeStruct((N, 1), jnp.int32),
            jax.ShapeDtypeStruct((1, C), jnp.float32),
            jax.ShapeDtypeStruct((1, 1), jnp.float32),
        ],
        scratch_shapes=[
            pltpu.VMEM((C, D), jnp.float32),
            pltpu.VMEM((C, 3 * D), jnp.float32),
        ],
        compiler_params=pltpu.CompilerParams(
            dimension_semantics=("arbitrary",)),
    )(z_flat, codebook)

    z_q_st = zq_flat.reshape(z.shape)
    mean_sq = loss_sum[0, 0] / (N * D)
    loss = BETA * mean_sq + mean_sq
    encoding_indices = idxo[:, 0]
    bin_count = bc[0].astype(jnp.int32)
    return z_q_st, loss, encoding_indices, bin_count


# speedup vs baseline: 1.2176x; 1.2176x over previous
import jax
import jax.numpy as jnp
from jax import lax
from jax.experimental import pallas as pl
from jax.experimental.pallas import tpu as pltpu

N = 36864
D = 64
C = 1024
R = 4096
NB = N // R
BETA = 0.25


def _vq_body(z_ref, cbt_ref, cb_ref, zq_ref, idx_ref, bc_ref, loss_ref,
             cbnt_ref, baug_ref):
    i = pl.program_id(0)

    @pl.when(i == 0)
    def _init():
        cbt = cbt_ref[...]
        n = jnp.sqrt(jnp.sum(cbt * cbt, axis=0, keepdims=True))
        cbnt_ref[...] = cbt / jnp.maximum(n, 1e-12)
        cb = cb_ref[...]
        h = cb.astype(jnp.bfloat16).astype(jnp.float32)
        mid = (cb - h).astype(jnp.bfloat16).astype(jnp.float32)
        baug_ref[:, 0:D] = h
        baug_ref[:, D:2 * D] = mid
        iota = lax.broadcasted_iota(jnp.int32, (C, 32), 0)
        baug_ref[:, 2 * D:2 * D + 32] = jnp.right_shift(iota, 8).astype(
            jnp.float32)
        baug_ref[:, 2 * D + 32:3 * D] = jnp.bitwise_and(iota, 255).astype(
            jnp.float32)
        bc_ref[...] = jnp.zeros_like(bc_ref)
        loss_ref[...] = jnp.zeros_like(loss_ref)

    zb = z_ref[...]
    zn = jnp.sqrt(jnp.sum(zb * zb, axis=1, keepdims=True))
    znorm = zb / jnp.maximum(zn, 1e-12)
    d = lax.dot_general(znorm, cbnt_ref[...], (((1,), (0,)), ((), ())),
                        precision=lax.Precision.DEFAULT,
                        preferred_element_type=jnp.float32)
    m = jnp.max(d, axis=1, keepdims=True)
    eqf = jnp.where(d == m, 1.0, 0.0).astype(jnp.float32)
    p = lax.dot_general(eqf, baug_ref[...], (((1,), (0,)), ((), ())),
                        precision=lax.Precision.DEFAULT,
                        preferred_element_type=jnp.float32)
    zq = p[:, 0:D] + p[:, D:2 * D]
    idx_ref[...] = (p[:, 2 * D:2 * D + 1] * 256.0
                    + p[:, 2 * D + 32:2 * D + 33]).astype(jnp.int32)
    zq_ref[...] = zb + (zq - zb)
    diff = zq - zb
    loss_ref[...] = loss_ref[...] + jnp.sum(diff * diff).reshape(1, 1)
    bc_ref[...] = bc_ref[...] + jnp.sum(eqf, axis=0, keepdims=True)


def kernel(z, codebook):
    z_flat = z.reshape(N, D)
    cbt = codebook.T
    zq_flat, idxf, bc, loss_sum = pl.pallas_call(
        _vq_body,
        grid=(NB,),
        in_specs=[
            pl.BlockSpec((R, D), lambda i: (i, 0)),
            pl.BlockSpec((D, C), lambda i: (0, 0)),
            pl.BlockSpec((C, D), lambda i: (0, 0)),
        ],
        out_specs=[
            pl.BlockSpec((R, D), lambda i: (i, 0)),
            pl.BlockSpec((R, 1), lambda i: (i, 0)),
            pl.BlockSpec((1, C), lambda i: (0, 0)),
            pl.BlockSpec((1, 1), lambda i: (0, 0)),
        ],
        out_shape=[
            jax.ShapeDtypeStruct((N, D), jnp.float32),
            jax.ShapeDtypeStruct((N, 1), jnp.int32),
            jax.ShapeDtypeStruct((1, C), jnp.float32),
            jax.ShapeDtypeStruct((1, 1), jnp.float32),
        ],
        scratch_shapes=[
            pltpu.VMEM((D, C), jnp.float32),
            pltpu.VMEM((C, 3 * D), jnp.float32),
        ],
        compiler_params=pltpu.CompilerParams(
            dimension_semantics=("arbitrary",)),
    )(z_flat, cbt, codebook)

    z_q_st = zq_flat.reshape(z.shape)
    mean_sq = loss_sum[0, 0] / (N * D)
    loss = BETA * mean_sq + mean_sq
    encoding_indices = idxf[:, 0]
    bin_count = bc[0].astype(jnp.int32)
    return z_q_st, loss, encoding_indices, bin_count


# speedup vs baseline: 1.2194x; 1.0015x over previous
import jax
import jax.numpy as jnp
from jax import lax
from jax.experimental import pallas as pl
from jax.experimental.pallas import tpu as pltpu

N = 36864
D = 64
C = 1024
R = 4608
NB = N // R
BETA = 0.25


def _vq_body(z_ref, cbt_ref, cb_ref, zq_ref, idx_ref, bc_ref, loss_ref,
             cbnt_ref, baug_ref):
    i = pl.program_id(0)

    @pl.when(i == 0)
    def _init():
        cbt = cbt_ref[...]
        n = jnp.sqrt(jnp.sum(cbt * cbt, axis=0, keepdims=True))
        cbnt_ref[...] = cbt / jnp.maximum(n, 1e-12)
        cb = cb_ref[...]
        h = cb.astype(jnp.bfloat16).astype(jnp.float32)
        mid = (cb - h).astype(jnp.bfloat16).astype(jnp.float32)
        baug_ref[:, 0:D] = h
        baug_ref[:, D:2 * D] = mid
        iota = lax.broadcasted_iota(jnp.int32, (C, 32), 0)
        baug_ref[:, 2 * D:2 * D + 32] = jnp.right_shift(iota, 8).astype(
            jnp.float32)
        baug_ref[:, 2 * D + 32:3 * D] = jnp.bitwise_and(iota, 255).astype(
            jnp.float32)
        bc_ref[...] = jnp.zeros_like(bc_ref)
        loss_ref[...] = jnp.zeros_like(loss_ref)

    zb = z_ref[...]
    zn = jnp.sqrt(jnp.sum(zb * zb, axis=1, keepdims=True))
    znorm = zb / jnp.maximum(zn, 1e-12)
    d = lax.dot_general(znorm, cbnt_ref[...], (((1,), (0,)), ((), ())),
                        precision=lax.Precision.DEFAULT,
                        preferred_element_type=jnp.float32)
    m = jnp.max(d, axis=1, keepdims=True)
    eqf = jnp.where(d == m, 1.0, 0.0).astype(jnp.float32)
    p = lax.dot_general(eqf, baug_ref[...], (((1,), (0,)), ((), ())),
                        precision=lax.Precision.DEFAULT,
                        preferred_element_type=jnp.float32)
    zq = p[:, 0:D] + p[:, D:2 * D]
    idx_ref[...] = (p[:, 2 * D:2 * D + 1] * 256.0
                    + p[:, 2 * D + 32:2 * D + 33]).astype(jnp.int32)
    zq_ref[...] = zb + (zq - zb)
    diff = zq - zb
    loss_ref[...] = loss_ref[...] + jnp.sum(diff * diff).reshape(1, 1)
    bc_ref[...] = bc_ref[...] + jnp.sum(eqf, axis=0, keepdims=True)


def kernel(z, codebook):
    z_flat = z.reshape(N, D)
    cbt = codebook.T
    zq_flat, idxf, bc, loss_sum = pl.pallas_call(
        _vq_body,
        grid=(NB,),
        in_specs=[
            pl.BlockSpec((R, D), lambda i: (i, 0)),
            pl.BlockSpec((D, C), lambda i: (0, 0)),
            pl.BlockSpec((C, D), lambda i: (0, 0)),
        ],
        out_specs=[
            pl.BlockSpec((R, D), lambda i: (i, 0)),
            pl.BlockSpec((R, 1), lambda i: (i, 0)),
            pl.BlockSpec((1, C), lambda i: (0, 0)),
            pl.BlockSpec((1, 1), lambda i: (0, 0)),
        ],
        out_shape=[
            jax.ShapeDtypeStruct((N, D), jnp.float32),
            jax.ShapeDtypeStruct((N, 1), jnp.int32),
            jax.ShapeDtypeStruct((1, C), jnp.float32),
            jax.ShapeDtypeStruct((1, 1), jnp.float32),
        ],
        scratch_shapes=[
            pltpu.VMEM((D, C), jnp.float32),
            pltpu.VMEM((C, 3 * D), jnp.float32),
        ],
        compiler_params=pltpu.CompilerParams(
            dimension_semantics=("arbitrary",)),
    )(z_flat, cbt, codebook)

    z_q_st = zq_flat.reshape(z.shape)
    mean_sq = loss_sum[0, 0] / (N * D)
    loss = BETA * mean_sq + mean_sq
    encoding_indices = idxf[:, 0]
    bin_count = bc[0].astype(jnp.int32)
    return z_q_st, loss, encoding_indices, bin_count
